# baseline (speedup 1.0000x reference)
import functools

import jax
import jax.numpy as jnp
from jax import lax
from jax.experimental import pallas as pl
from jax.experimental.pallas import tpu as pltpu
from jax.experimental.pallas import tpu_sc as plsc

N_COLS = 26
BATCH = 16384
VOCAB = 100000
DIM = 64

NC = 2
NS = 16
L = 16
NW = NC * NS
BPW = BATCH // NW
CHUNK = 128
NCH = BPW // CHUNK

_mesh = plsc.VectorSubcoreMesh(
    core_axis_name="c", subcore_axis_name="s", num_cores=NC, num_subcores=NS
)


@functools.partial(
    pl.kernel,
    out_type=jax.ShapeDtypeStruct((N_COLS, BATCH, DIM), jnp.float32),
    mesh=_mesh,
    scratch_types=[
        pltpu.VMEM((BPW,), jnp.int32),
        pltpu.VMEM((BPW, DIM), jnp.float32),
        pltpu.SemaphoreType.DMA,
    ],
    compiler_params=pltpu.CompilerParams(use_tc_tiling_on_sc=False),
)
def _embed(idx_hbm, tab_hbm, out_hbm, idx_v, rows_v, sem):
    wid = lax.axis_index("s") * NC + lax.axis_index("c")
    base = wid * BPW

    def col(c, carry):
        pltpu.sync_copy(idx_hbm.at[c, pl.ds(base, BPW)], idx_v)
        off = c * VOCAB
        for j in range(BPW // L):
            sl = pl.ds(j * L, L)
            idx_v[sl] = idx_v[sl] + off
        copies = [
            pltpu.async_copy(
                tab_hbm.at[idx_v.at[pl.ds(k * CHUNK, CHUNK)]],
                rows_v.at[pl.ds(k * CHUNK, CHUNK)],
                sem,
            )
            for k in range(NCH)
        ]
        for cp in copies:
            cp.wait()
        pltpu.sync_copy(rows_v, out_hbm.at[c, pl.ds(base, BPW)])
        return carry

    lax.fori_loop(0, N_COLS, col, None)


def kernel(indices, tables):
    tab_flat = tables.reshape(N_COLS * VOCAB, DIM)
    return _embed(indices, tab_flat)


# speedup vs baseline: 1.0157x; 1.0157x over previous
import functools

import jax
import jax.numpy as jnp
from jax import lax
from jax.experimental import pallas as pl
from jax.experimental.pallas import tpu as pltpu
from jax.experimental.pallas import tpu_sc as plsc

N_COLS = 26
BATCH = 16384
VOCAB = 100000
DIM = 64

NC = 2
NS = 16
NW = NC * NS
TOTAL = N_COLS * BATCH
RPW = TOTAL // NW
CHUNK = 128
BLK = 512
NCH = BLK // CHUNK
NBLK = RPW // BLK

_mesh = plsc.VectorSubcoreMesh(
    core_axis_name="c", subcore_axis_name="s", num_cores=NC, num_subcores=NS
)


@functools.partial(
    pl.kernel,
    out_type=jax.ShapeDtypeStruct((TOTAL, DIM), jnp.float32),
    mesh=_mesh,
    scratch_types=[
        pltpu.VMEM((RPW,), jnp.int32),
        pltpu.VMEM((BLK, DIM), jnp.float32),
        pltpu.VMEM((BLK, DIM), jnp.float32),
        pltpu.SemaphoreType.DMA,
        pltpu.SemaphoreType.DMA,
        pltpu.SemaphoreType.DMA,
        pltpu.SemaphoreType.DMA,
    ],
    compiler_params=pltpu.CompilerParams(use_tc_tiling_on_sc=False),
)
def _embed(idx_hbm, tab_hbm, out_hbm, idx_v, rows0, rows1, g0, g1, o0, o1):
    wid = lax.axis_index("s") * NC + lax.axis_index("c")
    base = wid * RPW
    rows = (rows0, rows1)
    gsem = (g0, g1)
    osem = (o0, o1)

    def fire_g(j, b):
        for k in range(NCH):
            pltpu.async_copy(
                tab_hbm.at[idx_v.at[pl.ds(j * BLK + k * CHUNK, CHUNK)]],
                rows[b].at[pl.ds(k * CHUNK, CHUNK)],
                gsem[b],
            )

    def drain_g(b):
        for k in range(NCH):
            pltpu.make_async_copy(
                tab_hbm.at[idx_v.at[pl.ds(k * CHUNK, CHUNK)]],
                rows[b].at[pl.ds(k * CHUNK, CHUNK)],
                gsem[b],
            ).wait()

    def fire_o(j, b):
        pltpu.async_copy(rows[b], out_hbm.at[pl.ds(base + j * BLK, BLK)], osem[b])

    def drain_o(b):
        pltpu.make_async_copy(
            rows[b], out_hbm.at[pl.ds(base, BLK)], osem[b]
        ).wait()

    pltpu.sync_copy(idx_hbm.at[pl.ds(base, RPW)], idx_v)
    fire_g(0, 0)
    fire_g(1, 1)

    @pl.loop(2, NBLK, step=2)
    def _(j0):
        for db in range(2):
            j = j0 + db
            drain_g(db)
            fire_o(j - 2, db)
            drain_o(db)
            fire_g(j, db)

    for db in range(2):
        drain_g(db)
        fire_o(NBLK - 2 + db, db)
    for db in range(2):
        drain_o(db)


def kernel(indices, tables):
    offs = (jnp.arange(N_COLS, dtype=jnp.int32) * VOCAB)[:, None]
    flat_idx = (indices + offs).reshape(TOTAL)
    tab_flat = tables.reshape(N_COLS * VOCAB, DIM)
    out_flat = _embed(flat_idx, tab_flat)
    return out_flat.reshape(N_COLS, BATCH, DIM)
